# baseline (speedup 1.0000x reference)
import functools

import jax
import jax.numpy as jnp
from jax import lax
from jax.experimental import pallas as pl
from jax.experimental.pallas import tpu as pltpu
from jax.experimental.pallas import tpu_sc as plsc

CYCLE = 168
T = 200
D = 64
B = 4096
TILED = CYCLE + T
NC = 2
NS = 16
NW = NC * NS
BPW = B // NW
LANES = 16


def _body(idx_hbm, data_hbm, off_hbm, out_hbm, table_v, idx_v, off_v, sem):
    wid = lax.axis_index("s") * NC + lax.axis_index("c")
    base = wid * BPW

    pltpu.sync_copy(data_hbm, table_v.at[pl.ds(0, CYCLE * D)])
    pltpu.sync_copy(data_hbm, table_v.at[pl.ds(CYCLE * D, CYCLE * D)])
    pltpu.sync_copy(
        data_hbm.at[pl.ds(0, (TILED - 2 * CYCLE) * D)],
        table_v.at[pl.ds(2 * CYCLE * D, (TILED - 2 * CYCLE) * D)],
    )

    pltpu.sync_copy(idx_hbm.at[pl.ds(base, BPW)], idx_v)
    pltpu.sync_copy(off_hbm, off_v)

    offv = off_v[...]
    for g in range(BPW // LANES):
        v = idx_v[pl.ds(g * LANES, LANES)]
        v = lax.rem(v + offv, jnp.int32(CYCLE))
        v = jnp.where(v < 0, v + jnp.int32(CYCLE), v)
        v = v * jnp.int32(D)
        for l in range(LANES):
            s = pl.multiple_of(v[l], D)
            o = pl.multiple_of((base + g * LANES + l) * (T * D), T * D)
            pltpu.make_async_copy(
                table_v.at[pl.ds(s, T * D)],
                out_hbm.at[pl.ds(o, T * D)],
                sem,
            ).start()

    def drain(b, _):
        pltpu.make_async_copy(
            table_v.at[pl.ds(0, T * D)],
            out_hbm.at[pl.ds(base * (T * D), T * D)],
            sem,
        ).wait()
        return _

    lax.fori_loop(0, BPW, drain, 0)


@jax.jit
def _run(index, data, length):
    off = jnp.full((LANES,), 1, dtype=jnp.int32) * (
        jnp.asarray(length, dtype=jnp.int32) - jnp.int32(T)
    )
    mesh = plsc.VectorSubcoreMesh(core_axis_name="c", subcore_axis_name="s")
    out = pl.kernel(
        _body,
        out_type=jax.ShapeDtypeStruct((B * T * D,), jnp.float32),
        mesh=mesh,
        scratch_types=[
            pltpu.VMEM((TILED * D,), jnp.float32),
            pltpu.VMEM((BPW,), jnp.int32),
            pltpu.VMEM((LANES,), jnp.int32),
            pltpu.SemaphoreType.DMA,
        ],
    )(index.reshape(B), data.reshape(CYCLE * D), off)
    return out.reshape(B, T, D)


def kernel(index, length, data):
    return _run(index, data, length)


# speedup vs baseline: 1.3225x; 1.3225x over previous
import functools

import jax
import jax.numpy as jnp
from jax import lax
from jax.experimental import pallas as pl
from jax.experimental.pallas import tpu as pltpu
from jax.experimental.pallas import tpu_sc as plsc

CYCLE = 168
T = 200
D = 64
B = 4096
TILED = CYCLE + T
NC = 2
NS = 16
NW = NC * NS
BPW = B // NW
LANES = 16


def _body(idx_hbm, data_hbm, off_hbm, out_hbm, table_v, idx_v, off_v, sem):
    wid = lax.axis_index("s") * NC + lax.axis_index("c")
    base = wid * BPW

    pltpu.sync_copy(data_hbm, table_v.at[pl.ds(0, CYCLE)])
    pltpu.sync_copy(data_hbm, table_v.at[pl.ds(CYCLE, CYCLE)])
    pltpu.sync_copy(
        data_hbm.at[pl.ds(0, TILED - 2 * CYCLE)],
        table_v.at[pl.ds(2 * CYCLE, TILED - 2 * CYCLE)],
    )

    pltpu.sync_copy(idx_hbm.at[pl.ds(base, BPW)], idx_v)
    pltpu.sync_copy(off_hbm, off_v)

    offv = off_v[...]
    for g in range(BPW // LANES):
        v = idx_v[pl.ds(g * LANES, LANES)]
        v = lax.rem(v + offv, jnp.int32(CYCLE))
        v = jnp.where(v < 0, v + jnp.int32(CYCLE), v)
        for l in range(LANES):
            s = v[l]
            pltpu.make_async_copy(
                table_v.at[pl.ds(s, T)], out_hbm.at[base + g * LANES + l], sem
            ).start()

    def drain(b, _):
        pltpu.make_async_copy(
            table_v.at[pl.ds(0, T)], out_hbm.at[base], sem
        ).wait()
        return _

    lax.fori_loop(0, BPW, drain, 0)


@jax.jit
def _run(index, data, length):
    off = jnp.full((LANES,), 1, dtype=jnp.int32) * (
        jnp.asarray(length, dtype=jnp.int32) - jnp.int32(T)
    )
    mesh = plsc.VectorSubcoreMesh(core_axis_name="c", subcore_axis_name="s")
    return pl.kernel(
        _body,
        out_type=jax.ShapeDtypeStruct((B, T, D), jnp.float32),
        mesh=mesh,
        scratch_types=[
            pltpu.VMEM((TILED, D), jnp.float32),
            pltpu.VMEM((BPW,), jnp.int32),
            pltpu.VMEM((LANES,), jnp.int32),
            pltpu.SemaphoreType.DMA,
        ],
    )(index, data, off)


def kernel(index, length, data):
    return _run(index, data, length)


# speedup vs baseline: 1.3273x; 1.0036x over previous
import functools

import jax
import jax.numpy as jnp
from jax import lax
from jax.experimental import pallas as pl
from jax.experimental.pallas import tpu as pltpu
from jax.experimental.pallas import tpu_sc as plsc

CYCLE = 168
T = 200
D = 64
B = 4096
TILED = CYCLE + T
NC = 2
NS = 16
NW = NC * NS
BPW = B // NW
LANES = 16


def _body(idx_hbm, data_hbm, off_hbm, out_hbm, table_v, idx_v, off_v, sem):
    wid = lax.axis_index("s") * NC + lax.axis_index("c")
    base = wid * BPW

    pltpu.sync_copy(data_hbm, table_v.at[pl.ds(0, CYCLE)])
    pltpu.sync_copy(data_hbm, table_v.at[pl.ds(CYCLE, CYCLE)])
    pltpu.sync_copy(
        data_hbm.at[pl.ds(0, TILED - 2 * CYCLE)],
        table_v.at[pl.ds(2 * CYCLE, TILED - 2 * CYCLE)],
    )

    pltpu.sync_copy(idx_hbm.at[pl.ds(base, BPW)], idx_v)
    pltpu.sync_copy(off_hbm, off_v)

    offv = off_v[...]
    for g in range(BPW // LANES):
        v = idx_v[pl.ds(g * LANES, LANES)]
        v = lax.rem(v + offv, jnp.int32(CYCLE))
        v = jnp.where(v < 0, v + jnp.int32(CYCLE), v)
        for l in range(LANES):
            s = v[l]
            pltpu.make_async_copy(
                table_v.at[pl.ds(s, T)], out_hbm.at[base + g * LANES + l], sem
            ).start()

    def drain(b, _):
        pltpu.make_async_copy(
            table_v.at[pl.ds(0, T)], out_hbm.at[base], sem
        ).wait()
        return _

    lax.fori_loop(0, BPW, drain, 0)


@jax.jit
def _run(index, data, length):
    off = jnp.full((LANES,), 1, dtype=jnp.int32) * (
        jnp.asarray(length, dtype=jnp.int32) - jnp.int32(T)
    )
    mesh = plsc.VectorSubcoreMesh(core_axis_name="c", subcore_axis_name="s")
    return pl.kernel(
        _body,
        out_type=jax.ShapeDtypeStruct((B, T, D), jnp.float32),
        mesh=mesh,
        compiler_params=pltpu.CompilerParams(use_tc_tiling_on_sc=True),
        scratch_types=[
            pltpu.VMEM((TILED, D), jnp.float32),
            pltpu.VMEM((BPW,), jnp.int32),
            pltpu.VMEM((LANES,), jnp.int32),
            pltpu.SemaphoreType.DMA,
        ],
    )(index, data, off)


def kernel(index, length, data):
    return _run(index, data, length)
